# Initial kernel scaffold; baseline (speedup 1.0000x reference)
import jax
import jax.numpy as jnp
from jax.experimental import pallas as pl


def kernel(x, edge_index, edge_weight, W, bias, prelu_a):
    raise NotImplementedError("write your pallas kernel here")


# speedup vs baseline: 3.6018x; 3.6018x over previous
import functools

import jax
import jax.numpy as jnp
from jax import lax
from jax.experimental import pallas as pl
from jax.experimental.pallas import tpu as pltpu
from jax.experimental.pallas import tpu_sc as plsc

N_NODES = 10000
N_PAD = 10112
N_CH = 128
LANES = 16
NC = 2
NS = 16
NW = NC * NS
EDGE_BATCH = 128


def _spmm_sc(x, col, row, w, zeros):
    e_total = col.shape[0]
    per_tile = e_total // NW
    batches = per_tile // EDGE_BATCH
    rows_per_tile = N_PAD // NS

    mesh = plsc.VectorSubcoreMesh(core_axis_name="c", subcore_axis_name="s")

    @functools.partial(
        pl.kernel,
        mesh=mesh,
        out_type=jax.ShapeDtypeStruct((NC, N_PAD, N_CH), jnp.float32),
        scratch_types=[
            pltpu.VMEM((EDGE_BATCH,), jnp.int32),
            pltpu.VMEM((EDGE_BATCH,), jnp.int32),
            pltpu.VMEM((EDGE_BATCH,), jnp.float32),
            pltpu.VMEM((EDGE_BATCH, N_CH), jnp.float32),
            pltpu.VMEM_SHARED((N_PAD, N_CH), jnp.float32),
            pltpu.SemaphoreType.DMA,
        ],
    )
    def spmm(x_hbm, col_hbm, row_hbm, w_hbm, z_hbm, out_hbm,
             col_v, row_v, w_v, rows_v, acc, sem):
        c = lax.axis_index("c")
        s = lax.axis_index("s")
        tid = c * NS + s

        pltpu.sync_copy(z_hbm.at[pl.ds(s * rows_per_tile, rows_per_tile)],
                        acc.at[pl.ds(s * rows_per_tile, rows_per_tile)])
        plsc.subcore_barrier()

        base0 = tid * per_tile

        def body(b, carry):
            base = pl.multiple_of(base0 + b * EDGE_BATCH, EDGE_BATCH)
            pltpu.sync_copy(col_hbm.at[pl.ds(base, EDGE_BATCH)], col_v)
            pltpu.sync_copy(row_hbm.at[pl.ds(base, EDGE_BATCH)], row_v)
            pltpu.sync_copy(w_hbm.at[pl.ds(base, EDGE_BATCH)], w_v)
            pltpu.async_copy(x_hbm.at[col_v], rows_v, sem).wait()

            def scale(g, carry2):
                wvec = w_v[pl.ds(g * LANES, LANES)]
                for l in range(LANES):
                    wj = wvec[l]
                    j = g * LANES + l
                    for kk in range(N_CH // LANES):
                        sl = (j, pl.ds(kk * LANES, LANES))
                        rows_v[sl] = rows_v[sl] * wj
                return carry2

            lax.fori_loop(0, EDGE_BATCH // LANES, scale, 0)
            pltpu.sync_copy(rows_v, acc.at[row_v], add=True)
            return carry

        lax.fori_loop(0, batches, body, 0)
        plsc.subcore_barrier()
        pltpu.sync_copy(acc.at[pl.ds(s * rows_per_tile, rows_per_tile)],
                        out_hbm.at[c, pl.ds(s * rows_per_tile, rows_per_tile)])

    return spmm(x, col, row, w, zeros)


def _combine_tc(p, W, bias, a):
    BLK = 400

    def body(p_ref, w_ref, b_ref, a_ref, o_ref):
        s = p_ref[0] + p_ref[1]
        y = lax.dot_general(s, w_ref[...], (((1,), (1,)), ((), ())),
                            preferred_element_type=jnp.float32)
        y = y + b_ref[...]
        alpha = a_ref[0]
        o_ref[...] = jnp.where(y >= 0, y, alpha * y)

    return pl.pallas_call(
        body,
        grid=(N_NODES // BLK,),
        in_specs=[
            pl.BlockSpec((2, BLK, N_CH), lambda i: (0, i, 0)),
            pl.BlockSpec((N_CH, N_CH), lambda i: (0, 0)),
            pl.BlockSpec((1, N_CH), lambda i: (0, 0)),
            pl.BlockSpec(memory_space=pltpu.SMEM),
        ],
        out_specs=pl.BlockSpec((BLK, N_CH), lambda i: (i, 0)),
        out_shape=jax.ShapeDtypeStruct((N_NODES, N_CH), jnp.float32),
    )(p, W, bias.reshape(1, N_CH), a.reshape(1))


def kernel(x, edge_index, edge_weight, W, bias, prelu_a):
    row = edge_index[0].astype(jnp.int32)
    col = edge_index[1].astype(jnp.int32)
    w = edge_weight.astype(jnp.float32)
    e = row.shape[0]
    chunk = NW * EDGE_BATCH
    e_pad = ((e + chunk - 1) // chunk) * chunk
    pad = e_pad - e
    if pad:
        row = jnp.concatenate([row, jnp.zeros((pad,), jnp.int32)])
        col = jnp.concatenate([col, jnp.zeros((pad,), jnp.int32)])
        w = jnp.concatenate([w, jnp.zeros((pad,), jnp.float32)])
    zeros = jnp.zeros((N_PAD, N_CH), jnp.float32)
    p = _spmm_sc(x, col, row, w, zeros)
    return _combine_tc(p, W, bias, prelu_a)
